# baseline (speedup 1.0000x reference)
import functools

import numpy as np
import jax
import jax.numpy as jnp
from jax import lax
from jax.experimental import pallas as pl
from jax.experimental.pallas import tpu as pltpu
from jax.experimental.pallas import tpu_sc as plsc

H = 4
AW = 16
CH = 128
NTILES = 32
F32 = jnp.float32
I32 = jnp.int32


def _ru(a, b):
    return -(-a // b) * b


def _tc_prep(xp, lin_w, u_w, c):
    R, Fin = xp.shape
    HO = lin_w.shape[1]
    O = HO // H
    YW = HO + 16
    BW = 80 if O == 16 else 16

    def body(x_ref, w_ref, u_ref, c_ref, y_ref, b_ref, sa_ref):
        xv = x_ref[...]
        xw = jnp.dot(xv, w_ref[...], preferred_element_type=F32)
        xu = jnp.dot(xv, u_ref[...], preferred_element_type=F32)
        cc = c_ref[...]
        a = jnp.exp(xu + cc)
        b = jnp.exp(-xu)
        ec = jnp.exp(cc - jnp.max(cc))
        sm = ec / jnp.sum(ec)
        sa = jnp.zeros((R, O), F32)
        for h in range(H):
            blk = xw[:, h * O:(h + 1) * O]
            y_ref[:, h * O:(h + 1) * O] = blk * a[:, h:h + 1]
            sa = sa + blk * sm[:, h:h + 1]
            y_ref[:, HO + 4 * h:HO + 4 * h + 4] = jnp.broadcast_to(
                a[:, h:h + 1], (R, 4))
            if O == 16:
                b_ref[:, 16 * h:16 * h + 16] = jnp.broadcast_to(
                    b[:, h:h + 1], (R, 16))
                b_ref[:, 64 + 4 * h:64 + 4 * h + 4] = jnp.broadcast_to(
                    b[:, h:h + 1], (R, 4))
            else:
                b_ref[:, 4 * h:4 * h + 4] = jnp.broadcast_to(
                    b[:, h:h + 1], (R, 4))
        sa_ref[...] = sa

    return pl.pallas_call(
        body,
        out_shape=[
            jax.ShapeDtypeStruct((R, YW), F32),
            jax.ShapeDtypeStruct((R, BW), F32),
            jax.ShapeDtypeStruct((R, O), F32),
        ],
    )(xp, lin_w, u_w, c.reshape(1, H))


def _tc_post(agg0, agg1, cnt0, cnt1, sa, bias, n_real, relu, bn):
    R, O = sa.shape

    def body(*refs):
        if bn is None:
            a0, a1, c0, c1, s_ref, b_ref, o_ref = refs
        else:
            a0, a1, c0, c1, s_ref, b_ref, g_ref, bb_ref, o_ref = refs
        aggw = a0[:R, :] + a1[:R, :]
        if O == 16:
            agg = aggw
        else:
            agg = (aggw[:, 0:4] + aggw[:, 4:8]
                   + aggw[:, 8:12] + aggw[:, 12:16])
        cnt = c0[:R, 0:1] + c1[:R, 0:1] + 1.0
        val = (agg + s_ref[...]) / cnt + b_ref[...]
        if relu:
            val = jnp.maximum(val, 0.0)
        row = lax.broadcasted_iota(I32, (R, O), 0)
        val = jnp.where(row < n_real, val, 0.0)
        if bn is not None:
            inv_n = 1.0 / n_real
            mu = jnp.sum(val, axis=0, keepdims=True) * inv_n
            ex2 = jnp.sum(val * val, axis=0, keepdims=True) * inv_n
            var = ex2 - mu * mu
            val = g_ref[...] * (val - mu) * lax.rsqrt(var + 1e-5) + bb_ref[...]
            val = jnp.where(row < n_real, val, 0.0)
        o_ref[...] = val

    args = [agg0, agg1, cnt0, cnt1, sa, bias.reshape(1, O)]
    if bn is not None:
        args += [bn[0].reshape(1, O), bn[1].reshape(1, O)]
    return pl.pallas_call(
        body,
        out_shape=jax.ShapeDtypeStruct((R, O), F32),
    )(*args)


def _tc_score(xp, w):
    R = xp.shape[0]

    def body(x_ref, w_ref, o_ref):
        wv = w_ref[...]
        nrm = lax.rsqrt(jnp.sum(wv * wv))
        s = jnp.dot(x_ref[...], wv, preferred_element_type=F32) * nrm
        o_ref[...] = jnp.tanh(s)

    return pl.pallas_call(
        body,
        out_shape=jax.ShapeDtypeStruct((R, 1), F32),
    )(xp, w)


def _tc_mlp(xp, p1, p2, p3, po, n_real):
    def body(x_ref, w1, b1, w2, b2, w3, b3, w4, b4, o_ref):
        z = x_ref[...]
        z = jnp.maximum(jnp.dot(z, w1[...], preferred_element_type=F32) + b1[...], 0.0)
        z = jnp.maximum(jnp.dot(z, w2[...], preferred_element_type=F32) + b2[...], 0.0)
        z = jnp.maximum(jnp.dot(z, w3[...], preferred_element_type=F32) + b3[...], 0.0)
        t = jnp.dot(z, w4[...], preferred_element_type=F32) + b4[...]
        o_ref[...] = (1.0 / (1.0 + jnp.exp(-t)))[:n_real, :]

    return pl.pallas_call(
        body,
        out_shape=jax.ShapeDtypeStruct((n_real, 1), F32),
    )(xp,
      p1['w'], p1['b'].reshape(1, -1),
      p2['w'], p2['b'].reshape(1, -1),
      p3['w'], p3['b'].reshape(1, -1),
      po['w'], po['b'].reshape(1, -1))


_MESH = plsc.VectorSubcoreMesh(core_axis_name="c", subcore_axis_name="s")
_SC_PARAMS = pltpu.CompilerParams(use_tc_tiling_on_sc=False,
                                  needs_layout_passes=False)


def _sc_edge(y, b, srcp, dstp, np_rows, O):
    R, YW = y.shape
    BW = b.shape[1]
    Ep = srcp.shape[0]
    cpt = Ep // CH // NTILES
    spt = cpt * CH
    rps = np_rows // 16
    n_grp = cpt // 2

    @functools.partial(
        pl.kernel, mesh=_MESH, compiler_params=_SC_PARAMS,
        out_type=[
            jax.ShapeDtypeStruct((np_rows, AW), F32),
            jax.ShapeDtypeStruct((np_rows, AW), F32),
        ],
        scratch_types=[
            pltpu.VMEM_SHARED((np_rows, AW), F32),
            pltpu.VMEM((spt,), I32),
            pltpu.VMEM((spt,), I32),
            pltpu.VMEM((CH, YW), F32),
            pltpu.VMEM((CH, YW), F32),
            pltpu.VMEM((CH, BW), F32),
            pltpu.VMEM((CH, BW), F32),
            pltpu.VMEM((CH, AW), F32),
            pltpu.VMEM((CH, AW), F32),
            pltpu.VMEM((CH,), I32),
            pltpu.VMEM((CH,), I32),
            pltpu.VMEM((rps, AW), F32),
            pltpu.SemaphoreType.DMA,
            pltpu.SemaphoreType.DMA,
            pltpu.SemaphoreType.DMA,
            pltpu.SemaphoreType.DMA,
            pltpu.SemaphoreType.DMA,
            pltpu.SemaphoreType.DMA,
        ],
    )
    def k(y_h, b_h, s_h, d_h, o0, o1, acc, sslab, dslab,
          yv0, yv1, bv0, bv1, mv0, mv1, db0, db1, zv,
          ys0, ys1, bs0, bs1, ss0, ss1):
        cid = lax.axis_index("c")
        sid = lax.axis_index("s")
        wid = cid * 16 + sid
        base_e = wid * spt
        yvs = (yv0, yv1)
        bvs = (bv0, bv1)
        mvs = (mv0, mv1)
        dbs = (db0, db1)
        ysem = (ys0, ys1)
        bsem = (bs0, bs1)
        ssem = (ss0, ss1)
        z16 = jnp.zeros((16,), F32)

        pltpu.sync_copy(s_h.at[pl.ds(base_e, spt)], sslab)
        pltpu.sync_copy(d_h.at[pl.ds(base_e, spt)], dslab)

        @plsc.parallel_loop(0, rps, unroll=8)
        def _(i):
            zv[i, :] = z16
        pltpu.sync_copy(zv, acc.at[pl.ds(sid * rps, rps)])
        plsc.subcore_barrier()

        def issue(ci, bidx):
            sidx = sslab.at[pl.ds(ci * CH, CH)]
            didx = dslab.at[pl.ds(ci * CH, CH)]
            pltpu.async_copy(y_h.at[sidx], yvs[bidx], ysem[bidx])
            pltpu.async_copy(b_h.at[didx], bvs[bidx], bsem[bidx])

        issue(0, 0)
        issue(1, 1)

        def group(cg, _):
            for bidx in range(2):
                ci = cg * 2 + bidx
                yv = yvs[bidx]
                bv = bvs[bidx]
                mv = mvs[bidx]
                db = dbs[bidx]
                sidx = sslab.at[pl.ds(ci * CH, CH)]
                pltpu.make_async_copy(y_h.at[sidx], yv, ysem[bidx]).wait()
                pltpu.make_async_copy(b_h.at[sidx], bv, bsem[bidx]).wait()

                @pl.when(cg > 0)
                def _():
                    pltpu.make_async_copy(mv, acc.at[db], ssem[bidx]).wait()

                @plsc.parallel_loop(0, CH // 16, unroll=4)
                def _(i):
                    db[pl.ds(i * 16, 16)] = dslab[pl.ds(ci * CH + i * 16, 16)]

                if O == 16:
                    @plsc.parallel_loop(0, CH, unroll=4)
                    def _(e):
                        aptn = yv[e, pl.ds(64, 16)]
                        bptn = bv[e, pl.ds(64, 16)]
                        sden = jnp.sum(aptn * bptn)
                        num = z16
                        for h in range(H):
                            num = num + (bv[e, pl.ds(h * 16, 16)]
                                         * yv[e, pl.ds(h * 16, 16)])
                        mv[e, :] = (num * 4.0) / jnp.full((16,), sden, F32)
                else:
                    @plsc.parallel_loop(0, CH, unroll=4)
                    def _(e):
                        yrow = yv[e, pl.ds(0, 16)]
                        aptn = yv[e, pl.ds(16, 16)]
                        bptn = bv[e, :]
                        sden = jnp.sum(aptn * bptn)
                        mv[e, :] = (yrow * bptn * 4.0) / jnp.full(
                            (16,), sden, F32)

                pltpu.async_copy(mv, acc.at[db], ssem[bidx], add=True)

                @pl.when(cg < n_grp - 1)
                def _():
                    issue(ci + 2, bidx)
            return 0
        lax.fori_loop(0, n_grp, group, 0)
        for bidx in range(2):
            pltpu.make_async_copy(mvs[bidx], acc.at[dbs[bidx]],
                                  ssem[bidx]).wait()
        plsc.subcore_barrier()

        @pl.when(cid == 0)
        def _():
            pltpu.sync_copy(acc.at[pl.ds(sid * rps, rps)],
                            o0.at[pl.ds(sid * rps, rps)])

        @pl.when(cid == 1)
        def _():
            pltpu.sync_copy(acc.at[pl.ds(sid * rps, rps)],
                            o1.at[pl.ds(sid * rps, rps)])

    return k(y, b, srcp, dstp)


def _sc_cnt(dstp, np_rows):
    Ep = dstp.shape[0]
    cpt = Ep // CH // NTILES
    spt = cpt * CH
    rps = np_rows // 16
    n_grp = cpt // 2

    @functools.partial(
        pl.kernel, mesh=_MESH, compiler_params=_SC_PARAMS,
        out_type=[
            jax.ShapeDtypeStruct((np_rows, AW), F32),
            jax.ShapeDtypeStruct((np_rows, AW), F32),
        ],
        scratch_types=[
            pltpu.VMEM_SHARED((np_rows, AW), F32),
            pltpu.VMEM((spt,), I32),
            pltpu.VMEM((CH, AW), F32),
            pltpu.VMEM((CH,), I32),
            pltpu.VMEM((CH,), I32),
            pltpu.VMEM((rps, AW), F32),
            pltpu.SemaphoreType.DMA,
            pltpu.SemaphoreType.DMA,
        ],
    )
    def k(d_h, o0, o1, acc, dslab, ov, db0, db1, zv, ss0, ss1):
        cid = lax.axis_index("c")
        sid = lax.axis_index("s")
        wid = cid * 16 + sid
        dbs = (db0, db1)
        ssem = (ss0, ss1)
        z16 = jnp.zeros((16,), F32)
        o16 = jnp.ones((16,), F32)

        pltpu.sync_copy(d_h.at[pl.ds(wid * spt, spt)], dslab)

        @plsc.parallel_loop(0, CH, unroll=8)
        def _(i):
            ov[i, :] = o16

        @plsc.parallel_loop(0, rps, unroll=8)
        def _(i):
            zv[i, :] = z16
        pltpu.sync_copy(zv, acc.at[pl.ds(sid * rps, rps)])
        plsc.subcore_barrier()

        def group(cg, _):
            for bidx in range(2):
                ci = cg * 2 + bidx
                db = dbs[bidx]

                @pl.when(cg > 0)
                def _():
                    pltpu.make_async_copy(ov, acc.at[db], ssem[bidx]).wait()

                @plsc.parallel_loop(0, CH // 16, unroll=4)
                def _(i):
                    db[pl.ds(i * 16, 16)] = dslab[pl.ds(ci * CH + i * 16, 16)]

                pltpu.async_copy(ov, acc.at[db], ssem[bidx], add=True)
            return 0
        lax.fori_loop(0, n_grp, group, 0)
        for bidx in range(2):
            pltpu.make_async_copy(ov, acc.at[dbs[bidx]], ssem[bidx]).wait()
        plsc.subcore_barrier()

        @pl.when(cid == 0)
        def _():
            pltpu.sync_copy(acc.at[pl.ds(sid * rps, rps)],
                            o0.at[pl.ds(sid * rps, rps)])

        @pl.when(cid == 1)
        def _():
            pltpu.sync_copy(acc.at[pl.ds(sid * rps, rps)],
                            o1.at[pl.ds(sid * rps, rps)])

    return k(dstp)


def _sc_pool(xp, permp, valsp, srcp, dstp, n_old, k_new):
    KP = permp.shape[0]
    Ep = srcp.shape[0]
    kcpt = KP // CH // NTILES
    spt = Ep // NTILES
    NT = _ru(n_old + 1, 16)

    @functools.partial(
        pl.kernel, mesh=_MESH, compiler_params=_SC_PARAMS,
        out_type=[
            jax.ShapeDtypeStruct((KP, 16), F32),
            jax.ShapeDtypeStruct((Ep,), I32),
            jax.ShapeDtypeStruct((Ep,), I32),
        ],
        scratch_types=[
            pltpu.VMEM((NT,), I32),
            pltpu.VMEM((KP,), I32),
            pltpu.VMEM((KP,), F32),
            pltpu.VMEM((CH, 16), F32),
            pltpu.VMEM((spt,), I32),
            pltpu.VMEM((spt,), I32),
            pltpu.VMEM((spt,), I32),
            pltpu.VMEM((spt,), I32),
            pltpu.SemaphoreType.DMA,
        ],
    )
    def k(x_h, p_h, v_h, s_h, d_h, xo, so, do,
          inv, pslab, vslab, xg, sslab, dslab, nss, nds, sem):
        cid = lax.axis_index("c")
        sid = lax.axis_index("s")
        wid = cid * 16 + sid
        base_e = wid * spt
        iot = lax.iota(I32, 16)
        kfull = jnp.full((16,), k_new, I32)

        pltpu.sync_copy(p_h, pslab)
        pltpu.sync_copy(v_h, vslab)
        pltpu.sync_copy(s_h.at[pl.ds(base_e, spt)], sslab)
        pltpu.sync_copy(d_h.at[pl.ds(base_e, spt)], dslab)

        @plsc.parallel_loop(0, NT // 16, unroll=8)
        def _(i):
            inv[pl.ds(i * 16, 16)] = kfull

        @plsc.parallel_loop(0, KP // 16, unroll=4)
        def _(i):
            pvals = pslab[pl.ds(i * 16, 16)]
            plsc.store_scatter(inv, [pvals],
                               jnp.full((16,), i * 16, I32) + iot)
        plsc.store_scatter(inv, [jnp.full((16,), n_old, I32)], kfull)

        for ci in range(kcpt):
            base = (wid * kcpt + ci) * CH
            pidx = pslab.at[pl.ds(base, CH)]
            pltpu.async_copy(x_h.at[pidx], xg, sem).wait()

            @plsc.parallel_loop(0, CH, unroll=4)
            def _(r):
                sv = plsc.load_gather(vslab, [jnp.full((16,), base + r, I32)])
                xg[r, :] = xg[r, :] * sv
            pltpu.sync_copy(xg, xo.at[pl.ds(base, CH)])

        @plsc.parallel_loop(0, spt // 16, unroll=4)
        def _(g):
            sv = sslab[pl.ds(g * 16, 16)]
            dv = dslab[pl.ds(g * 16, 16)]
            ns = plsc.load_gather(inv, [sv])
            nd = plsc.load_gather(inv, [dv])
            keep = jnp.logical_and(ns != kfull, nd != kfull)
            nss[pl.ds(g * 16, 16)] = jnp.where(keep, ns, kfull)
            nds[pl.ds(g * 16, 16)] = jnp.where(keep, nd, kfull)
        pltpu.sync_copy(nss, so.at[pl.ds(base_e, spt)])
        pltpu.sync_copy(nds, do.at[pl.ds(base_e, spt)])

    return k(xp, permp, valsp, srcp, dstp)


def _conv(xp, srcp, dstp, cnt0, cnt1, p, n_real, np_rows, relu, bn=None):
    y, b, sa = _tc_prep(xp, p['lin_w'], p['u_w'], p['c'])
    O = p['lin_w'].shape[1] // H
    agg0, agg1 = _sc_edge(y, b, srcp, dstp, np_rows, O)
    return _tc_post(agg0, agg1, cnt0, cnt1, sa, p['bias'], n_real, relu, bn)


def kernel(x, edge_index, params):
    N, F = x.shape
    E = edge_index.shape[1]
    src = edge_index[0].astype(I32)
    dst = edge_index[1].astype(I32)

    Ep = _ru(E, 2 * NTILES * CH)
    pad_e = Ep - E
    srcp = jnp.concatenate([src, jnp.full((pad_e,), N, I32)])
    dstp = jnp.concatenate([dst, jnp.full((pad_e,), N, I32)])
    xp = jnp.concatenate([x, jnp.zeros((1, F), F32)], axis=0)

    n = N
    for bname in ('b1', 'b2', 'b3'):
        p = params[bname]
        np_rows = _ru(n + 1, 128)
        cnt0, cnt1 = _sc_cnt(dstp, np_rows)
        xp = _conv(xp, srcp, dstp, cnt0, cnt1, p['c1'], n, np_rows, True)
        xp = _conv(xp, srcp, dstp, cnt0, cnt1, p['c2'], n, np_rows, True)
        xp = _conv(xp, srcp, dstp, cnt0, cnt1, p['c3'], n, np_rows, False)

        sc = _tc_score(xp, p['pool_w'].reshape(16, 1))
        scores = sc[:n, 0]
        k_new = int(np.ceil(0.5 * n))
        vals, perm = lax.top_k(scores, k_new)
        KP = _ru(k_new + 1, NTILES * CH)
        permp = jnp.concatenate([perm.astype(I32),
                                 jnp.full((KP - k_new,), n, I32)])
        valsp = jnp.concatenate([vals, jnp.zeros((KP - k_new,), F32)])
        xnew, srcp, dstp = _sc_pool(xp, permp, valsp, srcp, dstp, n, k_new)
        n = k_new
        xp = xnew[:n + 1]
        np_rows = _ru(n + 1, 128)
        cnt0, cnt1 = _sc_cnt(dstp, np_rows)
        xp = _conv(xp, srcp, dstp, cnt0, cnt1, p['c4'], n, np_rows, True,
                   bn=(p['bn_g'], p['bn_b']))

    for bname in ('b4', 'b5'):
        p = params[bname]
        xp = _conv(xp, srcp, dstp, cnt0, cnt1, p['c1'], n, np_rows, True)
        xp = _conv(xp, srcp, dstp, cnt0, cnt1, p['c2'], n, np_rows, True)
        xp = _conv(xp, srcp, dstp, cnt0, cnt1, p['c3'], n, np_rows, True)
        xp = _conv(xp, srcp, dstp, cnt0, cnt1, p['c4'], n, np_rows, True,
                   bn=(p['bn_g'], p['bn_b']))

    return _tc_mlp(xp, params['lin1'], params['lin2'], params['lin3'],
                   params['out'], n)
